# baseline (speedup 1.0000x reference)
import functools

import jax
import jax.numpy as jnp
from jax import lax
from jax.experimental import pallas as pl
from jax.experimental.pallas import tpu as pltpu
from jax.experimental.pallas import tpu_sc as plsc

SEQ = 200
D = 64
BATCH = 1024

BB = BATCH // 128
NTASK = SEQ * BB
NC, NS = 2, 16
NW = NC * NS
TPW = NTASK // NW
PAIRS = TPW // 2


def _make_kernel():
    mesh = plsc.VectorSubcoreMesh(core_axis_name="c", subcore_axis_name="s")

    @functools.partial(
        pl.kernel,
        out_type=jax.ShapeDtypeStruct((SEQ, D // 8, BB, 8, 128), jnp.float32),
        mesh=mesh,
        compiler_params=pltpu.CompilerParams(
            use_tc_tiling_on_sc=False, needs_layout_passes=False),
        scratch_types=[
            pltpu.VMEM((TPW, 128), jnp.int32),
            pltpu.VMEM((SEQ, D), jnp.float32),
            pltpu.VMEM((128, D), jnp.float32),
            pltpu.VMEM((128, D), jnp.float32),
            pltpu.VMEM((D // 8, 8, 128), jnp.float32),
            pltpu.VMEM((D // 8, 8, 128), jnp.float32),
            pltpu.SemaphoreType.DMA,
            pltpu.SemaphoreType.DMA,
            pltpu.SemaphoreType.DMA,
            pltpu.SemaphoreType.DMA,
        ],
    )
    def k(idx_hbm, word_hbm, pos_hbm, out_hbm,
          idx_v, pos_v, rows0, rows1, trans0, trans1, sg0, sg1, ss0, ss1):
        wid = lax.axis_index("s") * NC + lax.axis_index("c")
        t_base = wid * TPW
        pltpu.sync_copy(pos_hbm, pos_v)
        pltpu.sync_copy(idx_hbm.at[pl.ds(t_base, TPW)], idx_v)

        j = lax.iota(jnp.int32, 16)
        dim0_base = lax.shift_right_logical(j, 3)
        dim1_idx = lax.bitwise_and(j, 7)

        def gather(tl, rows_v, sem):
            return pltpu.make_async_copy(
                word_hbm.at[idx_v.at[tl]], rows_v, sem)

        def store_one(t, trans_v, db, sem):
            l = lax.div(t, BB)
            bb = lax.rem(t, BB)
            return pltpu.make_async_copy(
                trans_v.at[db], out_hbm.at[l, db, bb], sem)

        def store_start(t, trans_v, sem):
            for db in range(D // 8):
                store_one(t, trans_v, db, sem).start()

        def store_wait(t, trans_v, sem):
            for db in range(D // 8):
                store_one(t, trans_v, db, sem).wait()

        def scatter_add(t, rows_v, trans_v):
            l = lax.div(t, BB)
            p = [pos_v[l, pl.ds(c * 16, 16)] for c in range(D // 16)]
            d0 = [dim0_base + (2 * c) for c in range(D // 16)]

            def row_body(i, i_vec):
                for c in range(D // 16):
                    v = rows_v[i, pl.ds(c * 16, 16)] + p[c]
                    plsc.store_scatter(trans_v, [d0[c], dim1_idx, i_vec], v)
                return i_vec + 1

            lax.fori_loop(0, 128, row_body, j - j, unroll=4)

        gather(0, rows0, sg0).start()

        def pair_body(pp, _):
            tl0 = 2 * pp
            t0 = t_base + tl0
            gather(tl0 + 1, rows1, sg1).start()
            gather(tl0, rows0, sg0).wait()

            @pl.when(pp > 0)
            def _():
                store_wait(t0 - 2, trans0, ss0)

            scatter_add(t0, rows0, trans0)
            store_start(t0, trans0, ss0)

            @pl.when(pp + 1 < PAIRS)
            def _():
                gather(tl0 + 2, rows0, sg0).start()

            gather(tl0 + 1, rows1, sg1).wait()

            @pl.when(pp > 0)
            def _():
                store_wait(t0 - 1, trans1, ss1)

            scatter_add(t0 + 1, rows1, trans1)
            store_start(t0 + 1, trans1, ss1)
            return 0

        lax.fori_loop(0, PAIRS, pair_body, 0)
        store_wait(t_base + TPW - 2, trans0, ss0)
        store_wait(t_base + TPW - 1, trans1, ss1)

    return k


_kernel = _make_kernel()


@jax.jit
def kernel(inputs, word_table, pos_table):
    idx = inputs.astype(jnp.int32).T.reshape(SEQ * BB, 128)
    out5 = _kernel(idx, word_table, pos_table)
    return out5.transpose(2, 4, 0, 1, 3).reshape(BATCH, SEQ, D)


# speedup vs baseline: 1.7255x; 1.7255x over previous
import functools

import jax
import jax.numpy as jnp
from jax import lax
from jax.experimental import pallas as pl
from jax.experimental.pallas import tpu as pltpu
from jax.experimental.pallas import tpu_sc as plsc

SEQ = 200
D = 64
BATCH = 1024

BB = BATCH // 128
NTASK = SEQ * BB
NC, NS = 2, 16
NW = NC * NS
TPW = NTASK // NW
PAIRS = TPW // 2


def _make_kernel():
    mesh = plsc.VectorSubcoreMesh(core_axis_name="c", subcore_axis_name="s")

    @functools.partial(
        pl.kernel,
        out_type=jax.ShapeDtypeStruct((SEQ, D // 8, BB, 8, 128), jnp.float32),
        mesh=mesh,
        compiler_params=pltpu.CompilerParams(
            use_tc_tiling_on_sc=False, needs_layout_passes=False),
        scratch_types=[
            pltpu.VMEM((TPW, 128), jnp.int32),
            pltpu.VMEM((SEQ, D), jnp.float32),
            pltpu.VMEM((128, D), jnp.float32),
            pltpu.VMEM((128, D), jnp.float32),
            pltpu.VMEM((D, 129), jnp.float32),
            pltpu.VMEM((D, 129), jnp.float32),
            pltpu.SemaphoreType.DMA,
            pltpu.SemaphoreType.DMA,
            pltpu.SemaphoreType.DMA,
            pltpu.SemaphoreType.DMA,
        ],
    )
    def k(idx_hbm, word_hbm, pos_hbm, out_hbm,
          idx_v, pos_v, rows0, rows1, trans0, trans1, sg0, sg1, ss0, ss1):
        wid = lax.axis_index("s") * NC + lax.axis_index("c")
        t_base = wid * TPW
        pltpu.sync_copy(pos_hbm, pos_v)
        pltpu.sync_copy(idx_hbm.at[pl.ds(t_base, TPW)], idx_v)

        j = lax.iota(jnp.int32, 16)

        def gather(tl, rows_v, sem):
            return pltpu.make_async_copy(
                word_hbm.at[idx_v.at[tl]], rows_v, sem)

        def store_one(t, trans_v, db, sem):
            l = lax.div(t, BB)
            bb = lax.rem(t, BB)
            return pltpu.make_async_copy(
                trans_v.at[pl.ds(db * 8, 8), pl.ds(0, 128)],
                out_hbm.at[l, db, bb], sem)

        def store_start(t, trans_v, sem):
            for db in range(D // 8):
                store_one(t, trans_v, db, sem).start()

        def store_wait(t, trans_v, sem):
            for db in range(D // 8):
                store_one(t, trans_v, db, sem).wait()

        def scatter_add(t, rows_v, trans_v):
            l = lax.div(t, BB)
            p = [pos_v[l, pl.ds(c * 16, 16)] for c in range(D // 16)]
            dv = [j + (16 * c) for c in range(D // 16)]

            def row_body(i, i_vec):
                for c in range(D // 16):
                    v = rows_v[i, pl.ds(c * 16, 16)] + p[c]
                    plsc.store_scatter(trans_v, [dv[c], i_vec], v)
                return i_vec + 1

            lax.fori_loop(0, 128, row_body, j - j, unroll=4)

        gather(0, rows0, sg0).start()

        def pair_body(pp, _):
            tl0 = 2 * pp
            t0 = t_base + tl0
            gather(tl0 + 1, rows1, sg1).start()
            gather(tl0, rows0, sg0).wait()

            @pl.when(pp > 0)
            def _():
                store_wait(t0 - 2, trans0, ss0)

            scatter_add(t0, rows0, trans0)
            store_start(t0, trans0, ss0)

            @pl.when(pp + 1 < PAIRS)
            def _():
                gather(tl0 + 2, rows0, sg0).start()

            gather(tl0 + 1, rows1, sg1).wait()

            @pl.when(pp > 0)
            def _():
                store_wait(t0 - 1, trans1, ss1)

            scatter_add(t0 + 1, rows1, trans1)
            store_start(t0 + 1, trans1, ss1)
            return 0

        lax.fori_loop(0, PAIRS, pair_body, 0)
        store_wait(t_base + TPW - 2, trans0, ss0)
        store_wait(t_base + TPW - 1, trans1, ss1)

    return k


_kernel = _make_kernel()


@jax.jit
def kernel(inputs, word_table, pos_table):
    idx = inputs.astype(jnp.int32).T.reshape(SEQ * BB, 128)
    out5 = _kernel(idx, word_table, pos_table)
    return out5.transpose(2, 4, 0, 1, 3).reshape(BATCH, SEQ, D)


# speedup vs baseline: 2.6831x; 1.5550x over previous
import functools

import jax
import jax.numpy as jnp
from jax import lax
from jax.experimental import pallas as pl
from jax.experimental.pallas import tpu as pltpu
from jax.experimental.pallas import tpu_sc as plsc

SEQ = 200
D = 64
BATCH = 1024

BB = BATCH // 128
NTASK = SEQ * BB
NC, NS = 2, 16
NW = NC * NS
TPW = NTASK // NW
PAIRS = TPW // 2


def _make_kernel():
    mesh = plsc.VectorSubcoreMesh(core_axis_name="c", subcore_axis_name="s")

    @functools.partial(
        pl.kernel,
        out_type=jax.ShapeDtypeStruct((SEQ, D // 8, BB, 8, 128), jnp.float32),
        mesh=mesh,
        compiler_params=pltpu.CompilerParams(
            use_tc_tiling_on_sc=False, needs_layout_passes=False),
        scratch_types=[
            pltpu.VMEM((TPW, 128), jnp.int32),
            pltpu.VMEM((SEQ, D), jnp.float32),
            pltpu.VMEM((128, D), jnp.float32),
            pltpu.VMEM((128, D), jnp.float32),
            pltpu.VMEM((D, 129), jnp.float32),
            pltpu.VMEM((D, 129), jnp.float32),
            pltpu.SemaphoreType.DMA,
            pltpu.SemaphoreType.DMA,
            pltpu.SemaphoreType.DMA,
            pltpu.SemaphoreType.DMA,
        ],
    )
    def k(idx_hbm, word_hbm, pos_hbm, out_hbm,
          idx_v, pos_v, rows0, rows1, trans0, trans1, sg0, sg1, ss0, ss1):
        wid = lax.axis_index("s") * NC + lax.axis_index("c")
        t_base = wid * TPW
        pltpu.sync_copy(pos_hbm, pos_v)
        pltpu.sync_copy(idx_hbm.at[pl.ds(t_base, TPW)], idx_v)

        j = lax.iota(jnp.int32, 16)

        def gather(tl, rows_v, sem):
            return pltpu.make_async_copy(
                word_hbm.at[idx_v.at[tl]], rows_v, sem)

        def store_one(t, trans_v, db, sem):
            l = lax.div(t, BB)
            bb = lax.rem(t, BB)
            return pltpu.make_async_copy(
                trans_v.at[pl.ds(db * 8, 8), pl.ds(0, 128)],
                out_hbm.at[l, db, bb], sem)

        def store_start(t, trans_v, sem):
            for db in range(D // 8):
                store_one(t, trans_v, db, sem).start()

        def store_wait(t, trans_v, sem):
            for db in range(D // 8):
                store_one(t, trans_v, db, sem).wait()

        def scatter_add(t, rows_v, trans_v):
            l = lax.div(t, BB)
            p = [pos_v[l, pl.ds(c * 16, 16)] for c in range(D // 16)]
            dv = [j + (16 * c) for c in range(D // 16)]

            zeros = j - j

            @plsc.parallel_loop(0, 128, step=1, unroll=8)
            def row_body(i):
                for c in range(D // 16):
                    v = rows_v[i, pl.ds(c * 16, 16)] + p[c]
                    plsc.store_scatter(trans_v, [dv[c], zeros + i], v)

        gather(0, rows0, sg0).start()

        def pair_body(pp, _):
            tl0 = 2 * pp
            t0 = t_base + tl0
            gather(tl0 + 1, rows1, sg1).start()
            gather(tl0, rows0, sg0).wait()

            @pl.when(pp > 0)
            def _():
                store_wait(t0 - 2, trans0, ss0)

            scatter_add(t0, rows0, trans0)
            store_start(t0, trans0, ss0)

            @pl.when(pp + 1 < PAIRS)
            def _():
                gather(tl0 + 2, rows0, sg0).start()

            gather(tl0 + 1, rows1, sg1).wait()

            @pl.when(pp > 0)
            def _():
                store_wait(t0 - 1, trans1, ss1)

            scatter_add(t0 + 1, rows1, trans1)
            store_start(t0 + 1, trans1, ss1)
            return 0

        lax.fori_loop(0, PAIRS, pair_body, 0)
        store_wait(t_base + TPW - 2, trans0, ss0)
        store_wait(t_base + TPW - 1, trans1, ss1)

    return k


_kernel = _make_kernel()


@jax.jit
def kernel(inputs, word_table, pos_table):
    idx = inputs.astype(jnp.int32).T.reshape(SEQ * BB, 128)
    out5 = _kernel(idx, word_table, pos_table)
    return out5.transpose(2, 4, 0, 1, 3).reshape(BATCH, SEQ, D)


# speedup vs baseline: 2.8552x; 1.0641x over previous
import functools

import jax
import jax.numpy as jnp
from jax import lax
from jax.experimental import pallas as pl
from jax.experimental.pallas import tpu as pltpu
from jax.experimental.pallas import tpu_sc as plsc

SEQ = 200
D = 64
BATCH = 1024

BB = BATCH // 128
NTASK = SEQ * BB
NC, NS = 2, 16
NW = NC * NS
TPW = NTASK // NW
PAIRS = TPW // 2


def _make_kernel():
    mesh = plsc.VectorSubcoreMesh(core_axis_name="c", subcore_axis_name="s")

    @functools.partial(
        pl.kernel,
        out_type=jax.ShapeDtypeStruct((SEQ, D // 8, BB, 8, 128), jnp.float32),
        mesh=mesh,
        compiler_params=pltpu.CompilerParams(
            use_tc_tiling_on_sc=False, needs_layout_passes=False),
        scratch_types=[
            pltpu.VMEM((TPW, 128), jnp.int32),
            pltpu.VMEM((SEQ, D), jnp.float32),
            pltpu.VMEM((128, D), jnp.float32),
            pltpu.VMEM((128, D), jnp.float32),
            pltpu.VMEM((D, 129), jnp.float32),
            pltpu.VMEM((D, 129), jnp.float32),
            pltpu.SemaphoreType.DMA,
            pltpu.SemaphoreType.DMA,
            pltpu.SemaphoreType.DMA,
            pltpu.SemaphoreType.DMA,
        ],
    )
    def k(idx_hbm, word_hbm, pos_hbm, out_hbm,
          idx_v, pos_v, rows0, rows1, trans0, trans1, sg0, sg1, ss0, ss1):
        wid = lax.axis_index("s") * NC + lax.axis_index("c")
        t_base = wid * TPW
        pltpu.sync_copy(pos_hbm, pos_v)
        pltpu.sync_copy(idx_hbm.at[pl.ds(t_base, TPW)], idx_v)

        j = lax.iota(jnp.int32, 16)

        def gather(tl, rows_v, sem):
            return pltpu.make_async_copy(
                word_hbm.at[idx_v.at[tl]], rows_v, sem)

        def store_one(t, trans_v, db, sem):
            l = lax.div(t, BB)
            bb = lax.rem(t, BB)
            return pltpu.make_async_copy(
                trans_v.at[pl.ds(db * 8, 8), pl.ds(0, 128)],
                out_hbm.at[l, db, bb], sem)

        def store_start(t, trans_v, sem):
            for db in range(D // 8):
                store_one(t, trans_v, db, sem).start()

        def store_wait(t, trans_v, sem):
            for db in range(D // 8):
                store_one(t, trans_v, db, sem).wait()

        def scatter_add(t, rows_v, trans_v):
            l = lax.div(t, BB)
            p = [pos_v[l, pl.ds(c * 16, 16)] for c in range(D // 16)]
            dv = [j + (16 * c) for c in range(D // 16)]

            zeros = j - j

            @plsc.parallel_loop(0, 128, step=1, unroll=8)
            def row_body(i):
                for c in range(D // 16):
                    v = rows_v[i, pl.ds(c * 16, 16)] + p[c]
                    plsc.store_scatter(trans_v, [dv[c], zeros + i], v)

        gather(0, rows0, sg0).start()

        def pair_body(pp, _):
            tl0 = 2 * pp
            t0 = t_base + tl0
            gather(tl0 + 1, rows1, sg1).start()
            gather(tl0, rows0, sg0).wait()

            @pl.when(pp > 0)
            def _():
                store_wait(t0 - 2, trans0, ss0)

            scatter_add(t0, rows0, trans0)
            store_start(t0, trans0, ss0)

            @pl.when(pp + 1 < PAIRS)
            def _():
                gather(tl0 + 2, rows0, sg0).start()

            gather(tl0 + 1, rows1, sg1).wait()

            @pl.when(pp > 0)
            def _():
                store_wait(t0 - 1, trans1, ss1)

            scatter_add(t0 + 1, rows1, trans1)
            store_start(t0 + 1, trans1, ss1)
            return 0

        lax.fori_loop(0, PAIRS, pair_body, 0)
        store_wait(t_base + TPW - 2, trans0, ss0)
        store_wait(t_base + TPW - 1, trans1, ss1)

    return k


_kernel = _make_kernel()


@jax.jit
def kernel(inputs, word_table, pos_table):
    word2 = jnp.pad(word_table, ((0, 0), (0, D))).reshape(-1, D)
    idx = (inputs.astype(jnp.int32) * 2).T.reshape(SEQ * BB, 128)
    out5 = _kernel(idx, word2, pos_table)
    return out5.transpose(2, 4, 0, 1, 3).reshape(BATCH, SEQ, D)


# speedup vs baseline: 2.9253x; 1.0246x over previous
import functools

import jax
import jax.numpy as jnp
from jax import lax
from jax.experimental import pallas as pl
from jax.experimental.pallas import tpu as pltpu
from jax.experimental.pallas import tpu_sc as plsc

SEQ = 200
D = 64
BATCH = 1024

BB = BATCH // 128
NTASK = SEQ * BB
NC, NS = 2, 16
NW = NC * NS
TPW = NTASK // NW
PAIRS = TPW // 2


def _make_kernel():
    mesh = plsc.VectorSubcoreMesh(core_axis_name="c", subcore_axis_name="s")

    @functools.partial(
        pl.kernel,
        out_type=jax.ShapeDtypeStruct((SEQ, D // 8, BB, 8, 128), jnp.float32),
        mesh=mesh,
        compiler_params=pltpu.CompilerParams(
            use_tc_tiling_on_sc=False, needs_layout_passes=False),
        scratch_types=[
            pltpu.VMEM((TPW, 128), jnp.int32),
            pltpu.VMEM((SEQ, D), jnp.float32),
            pltpu.VMEM((128, D), jnp.float32),
            pltpu.VMEM((128, D), jnp.float32),
            pltpu.VMEM((D, 129), jnp.float32),
            pltpu.VMEM((D, 129), jnp.float32),
            pltpu.SemaphoreType.DMA,
            pltpu.SemaphoreType.DMA,
            pltpu.SemaphoreType.DMA,
            pltpu.SemaphoreType.DMA,
        ],
    )
    def k(idx_hbm, word_hbm, pos_hbm, out_hbm,
          idx_v, pos_v, rows0, rows1, trans0, trans1, sg0, sg1, ss0, ss1):
        wid = lax.axis_index("s") * NC + lax.axis_index("c")
        t_base = wid * TPW
        pltpu.sync_copy(pos_hbm, pos_v)
        pltpu.sync_copy(idx_hbm.at[pl.ds(t_base, TPW)], idx_v)

        j = lax.iota(jnp.int32, 16)

        def gather(tl, rows_v, sem):
            return pltpu.make_async_copy(
                word_hbm.at[idx_v.at[tl]], rows_v, sem)

        def store_one(t, trans_v, db, sem):
            l = lax.div(t, BB)
            bb = lax.rem(t, BB)
            return pltpu.make_async_copy(
                trans_v.at[pl.ds(db * 8, 8), pl.ds(0, 128)],
                out_hbm.at[l, db, bb], sem)

        def store_start(t, trans_v, sem):
            for db in range(D // 8):
                store_one(t, trans_v, db, sem).start()

        def store_wait(t, trans_v, sem):
            for db in range(D // 8):
                store_one(t, trans_v, db, sem).wait()

        def scatter_add(t, rows_v, trans_v):
            l = lax.div(t, BB)
            p = [pos_v[l, pl.ds(c * 16, 16)] for c in range(D // 16)]
            dv = [j + (16 * c) for c in range(D // 16)]

            zeros = j - j

            @plsc.parallel_loop(0, 128, step=1, unroll=16)
            def row_body(i):
                for c in range(D // 16):
                    v = rows_v[i, pl.ds(c * 16, 16)] + p[c]
                    plsc.store_scatter(trans_v, [dv[c], zeros + i], v)

        gather(0, rows0, sg0).start()

        def pair_body(pp, _):
            tl0 = 2 * pp
            t0 = t_base + tl0
            gather(tl0 + 1, rows1, sg1).start()
            gather(tl0, rows0, sg0).wait()

            @pl.when(pp > 0)
            def _():
                store_wait(t0 - 2, trans0, ss0)

            scatter_add(t0, rows0, trans0)
            store_start(t0, trans0, ss0)

            @pl.when(pp + 1 < PAIRS)
            def _():
                gather(tl0 + 2, rows0, sg0).start()

            gather(tl0 + 1, rows1, sg1).wait()

            @pl.when(pp > 0)
            def _():
                store_wait(t0 - 1, trans1, ss1)

            scatter_add(t0 + 1, rows1, trans1)
            store_start(t0 + 1, trans1, ss1)
            return 0

        lax.fori_loop(0, PAIRS, pair_body, 0)
        store_wait(t_base + TPW - 2, trans0, ss0)
        store_wait(t_base + TPW - 1, trans1, ss1)

    return k


_kernel = _make_kernel()


@jax.jit
def kernel(inputs, word_table, pos_table):
    word2 = jnp.pad(word_table, ((0, 0), (0, D))).reshape(-1, D)
    idx = (inputs.astype(jnp.int32) * 2).T.reshape(SEQ * BB, 128)
    out5 = _kernel(idx, word2, pos_table)
    return out5.transpose(2, 4, 0, 1, 3).reshape(BATCH, SEQ, D)
